# baseline (speedup 1.0000x reference)
import math

import jax
import jax.numpy as jnp
from jax.experimental import pallas as pl
from jax.experimental.pallas import tpu as pltpu

S, D, H, R = 2048, 1024, 16, 32
HR = H * R
BQ = 512
BK = 512
NBQ = S // BQ
NBK = S // BK
AW = 64
NEG = float(jnp.finfo(jnp.float32).min)
SCALE = math.log2(math.e) / math.sqrt(R)


def _qkv_kernel(x_ref, wq_ref, wk_ref, wv_ref, bq_ref, bk_ref, bv_ref,
                q_ref, k_ref, v_ref):
    x = x_ref[...]
    q = jax.lax.dot_general(x, wq_ref[...], (((1,), (0,)), ((), ())),
                            preferred_element_type=jnp.float32)
    k = jax.lax.dot_general(x, wk_ref[...], (((1,), (0,)), ((), ())),
                            preferred_element_type=jnp.float32)
    v = jax.lax.dot_general(x, wv_ref[...], (((1,), (0,)), ((), ())),
                            preferred_element_type=jnp.float32)
    q_ref[...] = (q + bq_ref[...]).astype(jnp.bfloat16)
    k_ref[...] = (k + bk_ref[...]).astype(jnp.bfloat16)
    v_ref[...] = (v + bv_ref[...]).astype(jnp.bfloat16)


def _attn_kernel(q_ref, k_ref, v_ref, wo_ref, bo_ref, out_ref, acc_ref):
    i = pl.program_id(0)
    j = pl.program_id(1)

    @pl.when(j == 0)
    def _init():
        acc_ref[...] = jnp.zeros_like(acc_ref)

    @pl.when(j <= i)
    def _compute():
        q = q_ref[...]
        k = k_ref[...]
        v = v_ref[...]
        row = i * BQ + jax.lax.broadcasted_iota(jnp.int32, (BQ, BK), 0)
        col = j * BK + jax.lax.broadcasted_iota(jnp.int32, (BQ, BK), 1)
        bias = jnp.where(row >= col, 0.0, NEG)
        aug = (jax.lax.broadcasted_iota(jnp.int32, (BK, AW - R), 1)
               == 0).astype(jnp.bfloat16)
        for h in range(H):
            qh = q[:, h * R:(h + 1) * R]
            kh = k[:, h * R:(h + 1) * R]
            vh = jnp.concatenate([v[:, h * R:(h + 1) * R], aug], axis=1)
            s = jax.lax.dot_general(qh, kh, (((1,), (1,)), ((), ())),
                                    preferred_element_type=jnp.float32)
            e = jnp.exp2(s + bias).astype(jnp.bfloat16)
            oh = jax.lax.dot_general(e, vh, (((1,), (0,)), ((), ())),
                                     preferred_element_type=jnp.float32)
            acc_ref[:, h * AW:(h + 1) * AW] = acc_ref[:, h * AW:(h + 1) * AW] + oh

    @pl.when(j == i)
    def _finalize():
        a = acc_ref[...]
        l = jnp.concatenate([a[:, h * AW + R:h * AW + R + 1] for h in range(H)],
                            axis=1)
        rl = 1.0 / l
        outs = []
        for h in range(H):
            blk = a[:, h * AW:h * AW + R]
            outs.append((blk * rl[:, h:h + 1]).astype(jnp.bfloat16))
        o = jnp.concatenate(outs, axis=1)
        out_ref[...] = jax.lax.dot_general(
            o, wo_ref[...], (((1,), (0,)), ((), ())),
            preferred_element_type=jnp.float32) + bo_ref[...]


def kernel(x, Wq, bq, Wk, bk, Wv, bv, Wo, bo):
    B = x.shape[0]
    x2 = x.reshape(S, D).astype(jnp.bfloat16)
    wq = (Wq * SCALE).astype(jnp.bfloat16)
    wk = Wk.astype(jnp.bfloat16)
    wv = Wv.astype(jnp.bfloat16)
    wo = Wo.astype(jnp.bfloat16)
    bq2 = (bq * SCALE).reshape(1, HR)
    bk2 = bk.reshape(1, HR)
    bv2 = bv.reshape(1, HR)
    bo2 = bo.reshape(1, D)

    q, k, v = pl.pallas_call(
        _qkv_kernel,
        grid=(NBQ,),
        in_specs=[
            pl.BlockSpec((BQ, D), lambda i: (i, 0)),
            pl.BlockSpec((D, HR), lambda i: (0, 0)),
            pl.BlockSpec((D, HR), lambda i: (0, 0)),
            pl.BlockSpec((D, HR), lambda i: (0, 0)),
            pl.BlockSpec((1, HR), lambda i: (0, 0)),
            pl.BlockSpec((1, HR), lambda i: (0, 0)),
            pl.BlockSpec((1, HR), lambda i: (0, 0)),
        ],
        out_specs=[
            pl.BlockSpec((BQ, HR), lambda i: (i, 0)),
            pl.BlockSpec((BQ, HR), lambda i: (i, 0)),
            pl.BlockSpec((BQ, HR), lambda i: (i, 0)),
        ],
        out_shape=[jax.ShapeDtypeStruct((S, HR), jnp.bfloat16)] * 3,
    )(x2, wq, wk, wv, bq2, bk2, bv2)

    out = pl.pallas_call(
        _attn_kernel,
        grid=(NBQ, NBK),
        in_specs=[
            pl.BlockSpec((BQ, HR), lambda i, j: (i, 0)),
            pl.BlockSpec((BK, HR), lambda i, j: (j, 0)),
            pl.BlockSpec((BK, HR), lambda i, j: (j, 0)),
            pl.BlockSpec((HR, D), lambda i, j: (0, 0)),
            pl.BlockSpec((1, D), lambda i, j: (0, 0)),
        ],
        out_specs=pl.BlockSpec((BQ, D), lambda i, j: (i, 0)),
        out_shape=jax.ShapeDtypeStruct((S, D), jnp.float32),
        scratch_shapes=[pltpu.VMEM((BQ, H * AW), jnp.float32)],
    )(q, k, v, wo, bo2)

    return out.reshape(B, S, D)


# speedup vs baseline: 1.0535x; 1.0535x over previous
import math

import jax
import jax.numpy as jnp
from jax.experimental import pallas as pl
from jax.experimental.pallas import tpu as pltpu

S, D, H, R = 2048, 1024, 16, 32
HR = H * R
BQ = 512
BK = 512
NBQ = S // BQ
NBK = S // BK
AW = 64
NEG = float(jnp.finfo(jnp.float32).min)
SCALE = math.log2(math.e) / math.sqrt(R)


def _qkv_kernel(x_ref, wq_ref, wk_ref, wv_ref, bq_ref, bk_ref, bv_ref,
                q_ref, k_ref, v_ref):
    x = x_ref[...]
    q = jax.lax.dot_general(x, wq_ref[...], (((1,), (0,)), ((), ())),
                            preferred_element_type=jnp.float32)
    k = jax.lax.dot_general(x, wk_ref[...], (((1,), (0,)), ((), ())),
                            preferred_element_type=jnp.float32)
    v = jax.lax.dot_general(x, wv_ref[...], (((1,), (0,)), ((), ())),
                            preferred_element_type=jnp.float32)
    q_ref[...] = (q + bq_ref[...]).astype(jnp.bfloat16)
    k_ref[...] = (k + bk_ref[...]).astype(jnp.bfloat16)
    v_ref[...] = (v + bv_ref[...]).astype(jnp.bfloat16)


def _attn_kernel(q_ref, k_ref, v_ref, wo_ref, bo_ref, out_ref, acc_ref):
    i = pl.program_id(0)
    j = pl.program_id(1)

    @pl.when(j == 0)
    def _init():
        acc_ref[...] = jnp.zeros_like(acc_ref)

    @pl.when(j <= i)
    def _compute():
        q = q_ref[...]
        k = k_ref[...]
        v = v_ref[...]
        row = i * BQ + jax.lax.broadcasted_iota(jnp.int32, (BQ, BK), 0)
        col = j * BK + jax.lax.broadcasted_iota(jnp.int32, (BQ, BK), 1)
        bias = jnp.where(row >= col, 0.0, NEG)
        aug = (jax.lax.broadcasted_iota(jnp.int32, (BK, AW - R), 1)
               == 0).astype(jnp.bfloat16)
        for h in range(H):
            qh = q[:, h * R:(h + 1) * R]
            kh = k[:, h * R:(h + 1) * R]
            vh = jnp.concatenate([v[:, h * R:(h + 1) * R], aug], axis=1)
            s = jax.lax.dot_general(qh, kh, (((1,), (1,)), ((), ())),
                                    preferred_element_type=jnp.float32)
            e = jnp.exp2(s + bias).astype(jnp.bfloat16)
            oh = jax.lax.dot_general(e, vh, (((1,), (0,)), ((), ())),
                                     preferred_element_type=jnp.float32)
            acc_ref[:, h * AW:(h + 1) * AW] = acc_ref[:, h * AW:(h + 1) * AW] + oh

    @pl.when(j == i)
    def _finalize():
        outs = []
        for h in range(H):
            blk = acc_ref[:, h * AW:(h + 1) * AW]
            outs.append((blk[:, :R] / blk[:, R:R + 1]).astype(jnp.bfloat16))
        o = jnp.concatenate(outs, axis=1)
        out_ref[...] = jax.lax.dot_general(
            o, wo_ref[...], (((1,), (0,)), ((), ())),
            preferred_element_type=jnp.float32) + bo_ref[...]


def kernel(x, Wq, bq, Wk, bk, Wv, bv, Wo, bo):
    B = x.shape[0]
    x2 = x.reshape(S, D).astype(jnp.bfloat16)
    wq = (Wq * SCALE).astype(jnp.bfloat16)
    wk = Wk.astype(jnp.bfloat16)
    wv = Wv.astype(jnp.bfloat16)
    wo = Wo.astype(jnp.bfloat16)
    bq2 = (bq * SCALE).reshape(1, HR)
    bk2 = bk.reshape(1, HR)
    bv2 = bv.reshape(1, HR)
    bo2 = bo.reshape(1, D)

    q, k, v = pl.pallas_call(
        _qkv_kernel,
        grid=(NBQ,),
        in_specs=[
            pl.BlockSpec((BQ, D), lambda i: (i, 0)),
            pl.BlockSpec((D, HR), lambda i: (0, 0)),
            pl.BlockSpec((D, HR), lambda i: (0, 0)),
            pl.BlockSpec((D, HR), lambda i: (0, 0)),
            pl.BlockSpec((1, HR), lambda i: (0, 0)),
            pl.BlockSpec((1, HR), lambda i: (0, 0)),
            pl.BlockSpec((1, HR), lambda i: (0, 0)),
        ],
        out_specs=[
            pl.BlockSpec((BQ, HR), lambda i: (i, 0)),
            pl.BlockSpec((BQ, HR), lambda i: (i, 0)),
            pl.BlockSpec((BQ, HR), lambda i: (i, 0)),
        ],
        out_shape=[jax.ShapeDtypeStruct((S, HR), jnp.bfloat16)] * 3,
    )(x2, wq, wk, wv, bq2, bk2, bv2)

    out = pl.pallas_call(
        _attn_kernel,
        grid=(NBQ, NBK),
        in_specs=[
            pl.BlockSpec((BQ, HR), lambda i, j: (i, 0)),
            pl.BlockSpec((BK, HR), lambda i, j: (j, 0)),
            pl.BlockSpec((BK, HR), lambda i, j: (j, 0)),
            pl.BlockSpec((HR, D), lambda i, j: (0, 0)),
            pl.BlockSpec((1, D), lambda i, j: (0, 0)),
        ],
        out_specs=pl.BlockSpec((BQ, D), lambda i, j: (i, 0)),
        out_shape=jax.ShapeDtypeStruct((S, D), jnp.float32),
        scratch_shapes=[pltpu.VMEM((BQ, H * AW), jnp.float32)],
    )(q, k, v, wo, bo2)

    return out.reshape(B, S, D)


# speedup vs baseline: 1.0609x; 1.0070x over previous
import math

import jax
import jax.numpy as jnp
from jax.experimental import pallas as pl
from jax.experimental.pallas import tpu as pltpu

S, D, H, R = 2048, 1024, 16, 32
HR = H * R
BQ = 512
BK = 512
NBQ = S // BQ
NBK = S // BK
AW = 64
NEG = float(jnp.finfo(jnp.float32).min)
SCALE = math.log2(math.e) / math.sqrt(R)


def _qkv_kernel(x_ref, wq_ref, wk_ref, wv_ref, bq_ref, bk_ref, bv_ref,
                q_ref, k_ref, v_ref):
    x = x_ref[...]
    q = jax.lax.dot_general(x, wq_ref[...], (((1,), (0,)), ((), ())),
                            preferred_element_type=jnp.float32)
    k = jax.lax.dot_general(x, wk_ref[...], (((1,), (0,)), ((), ())),
                            preferred_element_type=jnp.float32)
    v = jax.lax.dot_general(x, wv_ref[...], (((1,), (0,)), ((), ())),
                            preferred_element_type=jnp.float32)
    q_ref[...] = (q + bq_ref[...]).astype(jnp.bfloat16)
    k_ref[...] = (k + bk_ref[...]).astype(jnp.bfloat16)
    v_ref[...] = (v + bv_ref[...]).astype(jnp.bfloat16)


def _attn_kernel(q_ref, k_ref, v_ref, wo_ref, bo_ref, out_ref, acc_ref):
    i = pl.program_id(0)
    j = pl.program_id(1)

    @pl.when(j == 0)
    def _init():
        acc_ref[...] = jnp.zeros_like(acc_ref)

    @pl.when(j <= i)
    def _compute():
        q = q_ref[...]
        k = k_ref[...]
        v = v_ref[...]
        row = jax.lax.broadcasted_iota(jnp.int32, (BQ, BK), 0)
        col = jax.lax.broadcasted_iota(jnp.int32, (BQ, BK), 1)
        m01 = jnp.where((j == i) & (row < col), 0.0, 1.0).astype(jnp.bfloat16)
        aug = (jax.lax.broadcasted_iota(jnp.int32, (BK, AW - R), 1)
               == 0).astype(jnp.bfloat16)
        for h in range(H):
            qh = q[:, h * R:(h + 1) * R]
            kh = k[:, h * R:(h + 1) * R]
            vh = jnp.concatenate([v[:, h * R:(h + 1) * R], aug], axis=1)
            s = jax.lax.dot_general(qh, kh, (((1,), (1,)), ((), ())),
                                    preferred_element_type=jnp.float32)
            e = jnp.exp2(s).astype(jnp.bfloat16) * m01
            oh = jax.lax.dot_general(e, vh, (((1,), (0,)), ((), ())),
                                     preferred_element_type=jnp.float32)
            acc_ref[:, h * AW:(h + 1) * AW] = acc_ref[:, h * AW:(h + 1) * AW] + oh

    @pl.when(j == i)
    def _finalize():
        outs = []
        for h in range(H):
            blk = acc_ref[:, h * AW:(h + 1) * AW]
            outs.append((blk[:, :R] / blk[:, R:R + 1]).astype(jnp.bfloat16))
        o = jnp.concatenate(outs, axis=1)
        out_ref[...] = jax.lax.dot_general(
            o, wo_ref[...], (((1,), (0,)), ((), ())),
            preferred_element_type=jnp.float32) + bo_ref[...]


def kernel(x, Wq, bq, Wk, bk, Wv, bv, Wo, bo):
    B = x.shape[0]
    x2 = x.reshape(S, D).astype(jnp.bfloat16)
    wq = (Wq * SCALE).astype(jnp.bfloat16)
    wk = Wk.astype(jnp.bfloat16)
    wv = Wv.astype(jnp.bfloat16)
    wo = Wo.astype(jnp.bfloat16)
    bq2 = (bq * SCALE).reshape(1, HR)
    bk2 = bk.reshape(1, HR)
    bv2 = bv.reshape(1, HR)
    bo2 = bo.reshape(1, D)

    q, k, v = pl.pallas_call(
        _qkv_kernel,
        grid=(NBQ,),
        in_specs=[
            pl.BlockSpec((BQ, D), lambda i: (i, 0)),
            pl.BlockSpec((D, HR), lambda i: (0, 0)),
            pl.BlockSpec((D, HR), lambda i: (0, 0)),
            pl.BlockSpec((D, HR), lambda i: (0, 0)),
            pl.BlockSpec((1, HR), lambda i: (0, 0)),
            pl.BlockSpec((1, HR), lambda i: (0, 0)),
            pl.BlockSpec((1, HR), lambda i: (0, 0)),
        ],
        out_specs=[
            pl.BlockSpec((BQ, HR), lambda i: (i, 0)),
            pl.BlockSpec((BQ, HR), lambda i: (i, 0)),
            pl.BlockSpec((BQ, HR), lambda i: (i, 0)),
        ],
        out_shape=[jax.ShapeDtypeStruct((S, HR), jnp.bfloat16)] * 3,
    )(x2, wq, wk, wv, bq2, bk2, bv2)

    out = pl.pallas_call(
        _attn_kernel,
        grid=(NBQ, NBK),
        in_specs=[
            pl.BlockSpec((BQ, HR), lambda i, j: (i, 0)),
            pl.BlockSpec((BK, HR), lambda i, j: (j, 0)),
            pl.BlockSpec((BK, HR), lambda i, j: (j, 0)),
            pl.BlockSpec((HR, D), lambda i, j: (0, 0)),
            pl.BlockSpec((1, D), lambda i, j: (0, 0)),
        ],
        out_specs=pl.BlockSpec((BQ, D), lambda i, j: (i, 0)),
        out_shape=jax.ShapeDtypeStruct((S, D), jnp.float32),
        scratch_shapes=[pltpu.VMEM((BQ, H * AW), jnp.float32)],
    )(q, k, v, wo, bo2)

    return out.reshape(B, S, D)


# speedup vs baseline: 1.0902x; 1.0276x over previous
import math

import jax
import jax.numpy as jnp
from jax.experimental import pallas as pl
from jax.experimental.pallas import tpu as pltpu

S, D, H, R = 2048, 1024, 16, 32
HR = H * R
BQ = 512
BK = 512
NBQ = S // BQ
NBK = S // BK
AW = 64
SCALE = math.log2(math.e) / math.sqrt(R)


def _fused_kernel(x_ref, wq_ref, wk_ref, wv_ref, bq_ref, bk_ref, bv_ref,
                  wo_ref, bo_ref, out_ref, qblk_ref, kbuf_ref, vbuf_ref,
                  acc_ref):
    i = pl.program_id(0)
    j = pl.program_id(1)

    @pl.when(j == 0)
    def _qkv():
        x = x_ref[...]
        q = jax.lax.dot_general(x, wq_ref[...], (((1,), (0,)), ((), ())),
                                preferred_element_type=jnp.float32)
        k = jax.lax.dot_general(x, wk_ref[...], (((1,), (0,)), ((), ())),
                                preferred_element_type=jnp.float32)
        v = jax.lax.dot_general(x, wv_ref[...], (((1,), (0,)), ((), ())),
                                preferred_element_type=jnp.float32)
        qblk_ref[...] = (q + bq_ref[...]).astype(jnp.bfloat16)
        kbuf_ref[pl.ds(i * BQ, BQ), :] = (k + bk_ref[...]).astype(jnp.bfloat16)
        vbuf_ref[pl.ds(i * BQ, BQ), :] = (v + bv_ref[...]).astype(jnp.bfloat16)
        acc_ref[...] = jnp.zeros_like(acc_ref)

    @pl.when((j >= 1) & (j <= i + 1))
    def _attn():
        jj = j - 1
        q = qblk_ref[...]
        k = kbuf_ref[pl.ds(jj * BK, BK), :]
        v = vbuf_ref[pl.ds(jj * BK, BK), :]
        row = jax.lax.broadcasted_iota(jnp.int32, (BQ, BK), 0)
        col = jax.lax.broadcasted_iota(jnp.int32, (BQ, BK), 1)
        m01 = jnp.where((jj == i) & (row < col), 0.0, 1.0).astype(jnp.bfloat16)
        aug = (jax.lax.broadcasted_iota(jnp.int32, (BK, AW - R), 1)
               == 0).astype(jnp.bfloat16)
        for h in range(H):
            qh = q[:, h * R:(h + 1) * R]
            kh = k[:, h * R:(h + 1) * R]
            vh = jnp.concatenate([v[:, h * R:(h + 1) * R], aug], axis=1)
            s = jax.lax.dot_general(qh, kh, (((1,), (1,)), ((), ())),
                                    preferred_element_type=jnp.float32)
            e = jnp.exp2(s).astype(jnp.bfloat16) * m01
            oh = jax.lax.dot_general(e, vh, (((1,), (0,)), ((), ())),
                                     preferred_element_type=jnp.float32)
            acc_ref[:, h * AW:(h + 1) * AW] = acc_ref[:, h * AW:(h + 1) * AW] + oh

    @pl.when(j == i + 1)
    def _finalize():
        outs = []
        for h in range(H):
            blk = acc_ref[:, h * AW:(h + 1) * AW]
            outs.append((blk[:, :R] / blk[:, R:R + 1]).astype(jnp.bfloat16))
        o = jnp.concatenate(outs, axis=1)
        out_ref[...] = jax.lax.dot_general(
            o, wo_ref[...], (((1,), (0,)), ((), ())),
            preferred_element_type=jnp.float32) + bo_ref[...]


def kernel(x, Wq, bq, Wk, bk, Wv, bv, Wo, bo):
    B = x.shape[0]
    x2 = x.reshape(S, D).astype(jnp.bfloat16)
    wq = (Wq * SCALE).astype(jnp.bfloat16)
    wk = Wk.astype(jnp.bfloat16)
    wv = Wv.astype(jnp.bfloat16)
    wo = Wo.astype(jnp.bfloat16)
    bq2 = (bq * SCALE).reshape(1, HR)
    bk2 = bk.reshape(1, HR)
    bv2 = bv.reshape(1, HR)
    bo2 = bo.reshape(1, D)

    out = pl.pallas_call(
        _fused_kernel,
        grid=(NBQ, NBK + 1),
        in_specs=[
            pl.BlockSpec((BQ, D), lambda i, j: (i, 0)),
            pl.BlockSpec((D, HR), lambda i, j: (0, 0)),
            pl.BlockSpec((D, HR), lambda i, j: (0, 0)),
            pl.BlockSpec((D, HR), lambda i, j: (0, 0)),
            pl.BlockSpec((1, HR), lambda i, j: (0, 0)),
            pl.BlockSpec((1, HR), lambda i, j: (0, 0)),
            pl.BlockSpec((1, HR), lambda i, j: (0, 0)),
            pl.BlockSpec((HR, D), lambda i, j: (0, 0)),
            pl.BlockSpec((1, D), lambda i, j: (0, 0)),
        ],
        out_specs=pl.BlockSpec((BQ, D), lambda i, j: (i, 0)),
        out_shape=jax.ShapeDtypeStruct((S, D), jnp.float32),
        scratch_shapes=[
            pltpu.VMEM((BQ, HR), jnp.bfloat16),
            pltpu.VMEM((S, HR), jnp.bfloat16),
            pltpu.VMEM((S, HR), jnp.bfloat16),
            pltpu.VMEM((BQ, H * AW), jnp.float32),
        ],
    )(x2, wq, wk, wv, bq2, bk2, bv2, wo, bo2)

    return out.reshape(B, S, D)


# speedup vs baseline: 1.1618x; 1.0657x over previous
import math

import jax
import jax.numpy as jnp
from jax.experimental import pallas as pl
from jax.experimental.pallas import tpu as pltpu

S, D, H, R = 2048, 1024, 16, 32
HR = H * R
BQ = 512
BK = 512
NBQ = S // BQ
NBK = S // BK
AW = 64
SCALE = math.log2(math.e) / math.sqrt(R)


def _fused_kernel(x_ref, wq_ref, wk_ref, wv_ref, bq_ref, bk_ref, bv_ref,
                  wo_ref, bo_ref, out_ref, qblk_ref, kbuf_ref, vbuf_ref,
                  acc_ref):
    i = pl.program_id(0)
    j = pl.program_id(1)

    @pl.when(j == 0)
    def _qkv():
        x = x_ref[...]
        q = jax.lax.dot_general(x, wq_ref[...], (((1,), (0,)), ((), ())),
                                preferred_element_type=jnp.float32)
        k = jax.lax.dot_general(x, wk_ref[...], (((1,), (0,)), ((), ())),
                                preferred_element_type=jnp.float32)
        v = jax.lax.dot_general(x, wv_ref[...], (((1,), (0,)), ((), ())),
                                preferred_element_type=jnp.float32)
        qblk_ref[...] = (q + bq_ref[...]).astype(jnp.bfloat16)
        kbuf_ref[pl.ds(i * BQ, BQ), :] = (k + bk_ref[...]).astype(jnp.bfloat16)
        vbuf_ref[pl.ds(i * BQ, BQ), :] = (v + bv_ref[...]).astype(jnp.bfloat16)
        acc_ref[...] = jnp.zeros_like(acc_ref)

    @pl.when((j >= 1) & (j <= i))
    def _attn_offdiag():
        jj = j - 1
        q = qblk_ref[...]
        k = kbuf_ref[pl.ds(jj * BK, BK), :]
        v = vbuf_ref[pl.ds(jj * BK, BK), :]
        aug = (jax.lax.broadcasted_iota(jnp.int32, (BK, AW - R), 1)
               == 0).astype(jnp.bfloat16)
        for h in range(H):
            qh = q[:, h * R:(h + 1) * R]
            kh = k[:, h * R:(h + 1) * R]
            vh = jnp.concatenate([v[:, h * R:(h + 1) * R], aug], axis=1)
            s = jax.lax.dot_general(qh, kh, (((1,), (1,)), ((), ())),
                                    preferred_element_type=jnp.float32)
            e = jnp.exp2(s).astype(jnp.bfloat16)
            oh = jax.lax.dot_general(e, vh, (((1,), (0,)), ((), ())),
                                     preferred_element_type=jnp.float32)
            acc_ref[:, h * AW:(h + 1) * AW] = acc_ref[:, h * AW:(h + 1) * AW] + oh

    @pl.when(j == i + 1)
    def _attn_diag_and_finalize():
        q = qblk_ref[...]
        k = kbuf_ref[pl.ds(i * BK, BK), :]
        v = vbuf_ref[pl.ds(i * BK, BK), :]
        row = jax.lax.broadcasted_iota(jnp.int32, (BQ, BK), 0)
        col = jax.lax.broadcasted_iota(jnp.int32, (BQ, BK), 1)
        m01 = (row >= col).astype(jnp.bfloat16)
        aug = (jax.lax.broadcasted_iota(jnp.int32, (BK, AW - R), 1)
               == 0).astype(jnp.bfloat16)
        outs = []
        for h in range(H):
            qh = q[:, h * R:(h + 1) * R]
            kh = k[:, h * R:(h + 1) * R]
            vh = jnp.concatenate([v[:, h * R:(h + 1) * R], aug], axis=1)
            s = jax.lax.dot_general(qh, kh, (((1,), (1,)), ((), ())),
                                    preferred_element_type=jnp.float32)
            e = jnp.exp2(s).astype(jnp.bfloat16) * m01
            oh = jax.lax.dot_general(e, vh, (((1,), (0,)), ((), ())),
                                     preferred_element_type=jnp.float32)
            blk = acc_ref[:, h * AW:(h + 1) * AW] + oh
            outs.append((blk[:, :R] / blk[:, R:R + 1]).astype(jnp.bfloat16))
        o = jnp.concatenate(outs, axis=1)
        out_ref[...] = jax.lax.dot_general(
            o, wo_ref[...], (((1,), (0,)), ((), ())),
            preferred_element_type=jnp.float32) + bo_ref[...]


def kernel(x, Wq, bq, Wk, bk, Wv, bv, Wo, bo):
    B = x.shape[0]
    x2 = x.reshape(S, D).astype(jnp.bfloat16)
    wq = (Wq * SCALE).astype(jnp.bfloat16)
    wk = Wk.astype(jnp.bfloat16)
    wv = Wv.astype(jnp.bfloat16)
    wo = Wo.astype(jnp.bfloat16)
    bq2 = (bq * SCALE).reshape(1, HR)
    bk2 = bk.reshape(1, HR)
    bv2 = bv.reshape(1, HR)
    bo2 = bo.reshape(1, D)

    out = pl.pallas_call(
        _fused_kernel,
        grid=(NBQ, NBK + 1),
        in_specs=[
            pl.BlockSpec((BQ, D), lambda i, j: (i, 0)),
            pl.BlockSpec((D, HR), lambda i, j: (0, 0)),
            pl.BlockSpec((D, HR), lambda i, j: (0, 0)),
            pl.BlockSpec((D, HR), lambda i, j: (0, 0)),
            pl.BlockSpec((1, HR), lambda i, j: (0, 0)),
            pl.BlockSpec((1, HR), lambda i, j: (0, 0)),
            pl.BlockSpec((1, HR), lambda i, j: (0, 0)),
            pl.BlockSpec((HR, D), lambda i, j: (0, 0)),
            pl.BlockSpec((1, D), lambda i, j: (0, 0)),
        ],
        out_specs=pl.BlockSpec((BQ, D), lambda i, j: (i, 0)),
        out_shape=jax.ShapeDtypeStruct((S, D), jnp.float32),
        scratch_shapes=[
            pltpu.VMEM((BQ, HR), jnp.bfloat16),
            pltpu.VMEM((S, HR), jnp.bfloat16),
            pltpu.VMEM((S, HR), jnp.bfloat16),
            pltpu.VMEM((BQ, H * AW), jnp.float32),
        ],
    )(x2, wq, wk, wv, bq2, bk2, bv2, wo, bo2)

    return out.reshape(B, S, D)


# speedup vs baseline: 1.1650x; 1.0027x over previous
import math

import jax
import jax.numpy as jnp
from jax.experimental import pallas as pl
from jax.experimental.pallas import tpu as pltpu

S, D, H, R = 2048, 1024, 16, 32
HR = H * R
BQ = 512
BK = 512
NBQ = S // BQ
NBK = S // BK
AW = 64
SCALE = math.log2(math.e) / math.sqrt(R)


def _fused_kernel(x_ref, wq_ref, wk_ref, wv_ref, bq_ref, bk_ref, bv_ref,
                  wo_ref, bo_ref, out_ref, qblk_ref, kbuf_ref, vbuf_ref,
                  acc_ref):
    i = pl.program_id(0)
    j = pl.program_id(1)

    @pl.when(j == 0)
    def _qkv():
        x = x_ref[...]
        q = jax.lax.dot_general(x, wq_ref[...], (((1,), (0,)), ((), ())),
                                preferred_element_type=jnp.float32)
        k = jax.lax.dot_general(x, wk_ref[...], (((1,), (0,)), ((), ())),
                                preferred_element_type=jnp.float32)
        v = jax.lax.dot_general(x, wv_ref[...], (((1,), (0,)), ((), ())),
                                preferred_element_type=jnp.float32)
        qblk_ref[...] = (q + bq_ref[...]).astype(jnp.bfloat16)
        kbuf_ref[pl.ds(i * BQ, BQ), :] = (k + bk_ref[...]).astype(jnp.bfloat16)
        vbuf_ref[pl.ds(i * BQ, BQ), :] = (v + bv_ref[...]).astype(jnp.bfloat16)
        acc_ref[...] = jnp.zeros_like(acc_ref)

    @pl.when((j >= 1) & (j <= i))
    def _attn_offdiag():
        jj = j - 1
        aug = (jax.lax.broadcasted_iota(jnp.int32, (BK, AW - R), 1)
               == 0).astype(jnp.bfloat16)
        for h in range(H):
            qh = qblk_ref[:, h * R:(h + 1) * R]
            kh = kbuf_ref[pl.ds(jj * BK, BK), h * R:(h + 1) * R]
            vh = jnp.concatenate(
                [vbuf_ref[pl.ds(jj * BK, BK), h * R:(h + 1) * R], aug], axis=1)
            s = jax.lax.dot_general(qh, kh, (((1,), (1,)), ((), ())),
                                    preferred_element_type=jnp.float32)
            e = jnp.exp2(s).astype(jnp.bfloat16)
            oh = jax.lax.dot_general(e, vh, (((1,), (0,)), ((), ())),
                                     preferred_element_type=jnp.float32)
            acc_ref[:, h * AW:(h + 1) * AW] = acc_ref[:, h * AW:(h + 1) * AW] + oh

    @pl.when(j == i + 1)
    def _attn_diag_and_finalize():
        row = jax.lax.broadcasted_iota(jnp.int32, (BQ, BK), 0)
        col = jax.lax.broadcasted_iota(jnp.int32, (BQ, BK), 1)
        m01 = (row >= col).astype(jnp.bfloat16)
        aug = (jax.lax.broadcasted_iota(jnp.int32, (BK, AW - R), 1)
               == 0).astype(jnp.bfloat16)
        outs = []
        for h in range(H):
            qh = qblk_ref[:, h * R:(h + 1) * R]
            kh = kbuf_ref[pl.ds(i * BK, BK), h * R:(h + 1) * R]
            vh = jnp.concatenate(
                [vbuf_ref[pl.ds(i * BK, BK), h * R:(h + 1) * R], aug], axis=1)
            s = jax.lax.dot_general(qh, kh, (((1,), (1,)), ((), ())),
                                    preferred_element_type=jnp.float32)
            e = jnp.exp2(s).astype(jnp.bfloat16) * m01
            oh = jax.lax.dot_general(e, vh, (((1,), (0,)), ((), ())),
                                     preferred_element_type=jnp.float32)
            blk = acc_ref[:, h * AW:(h + 1) * AW] + oh
            outs.append((blk[:, :R] / blk[:, R:R + 1]).astype(jnp.bfloat16))
        o = jnp.concatenate(outs, axis=1)
        out_ref[...] = jax.lax.dot_general(
            o, wo_ref[...], (((1,), (0,)), ((), ())),
            preferred_element_type=jnp.float32) + bo_ref[...]


def kernel(x, Wq, bq, Wk, bk, Wv, bv, Wo, bo):
    B = x.shape[0]
    x2 = x.reshape(S, D).astype(jnp.bfloat16)
    wq = (Wq * SCALE).astype(jnp.bfloat16)
    wk = Wk.astype(jnp.bfloat16)
    wv = Wv.astype(jnp.bfloat16)
    wo = Wo.astype(jnp.bfloat16)
    bq2 = (bq * SCALE).reshape(1, HR)
    bk2 = bk.reshape(1, HR)
    bv2 = bv.reshape(1, HR)
    bo2 = bo.reshape(1, D)

    out = pl.pallas_call(
        _fused_kernel,
        grid=(NBQ, NBK + 1),
        in_specs=[
            pl.BlockSpec((BQ, D), lambda i, j: (i, 0)),
            pl.BlockSpec((D, HR), lambda i, j: (0, 0)),
            pl.BlockSpec((D, HR), lambda i, j: (0, 0)),
            pl.BlockSpec((D, HR), lambda i, j: (0, 0)),
            pl.BlockSpec((1, HR), lambda i, j: (0, 0)),
            pl.BlockSpec((1, HR), lambda i, j: (0, 0)),
            pl.BlockSpec((1, HR), lambda i, j: (0, 0)),
            pl.BlockSpec((HR, D), lambda i, j: (0, 0)),
            pl.BlockSpec((1, D), lambda i, j: (0, 0)),
        ],
        out_specs=pl.BlockSpec((BQ, D), lambda i, j: (i, 0)),
        out_shape=jax.ShapeDtypeStruct((S, D), jnp.float32),
        scratch_shapes=[
            pltpu.VMEM((BQ, HR), jnp.bfloat16),
            pltpu.VMEM((S, HR), jnp.bfloat16),
            pltpu.VMEM((S, HR), jnp.bfloat16),
            pltpu.VMEM((BQ, H * AW), jnp.float32),
        ],
    )(x2, wq, wk, wv, bq2, bk2, bv2, wo, bo2)

    return out.reshape(B, S, D)
